# baseline (speedup 1.0000x reference)
import functools

import jax
import jax.numpy as jnp
from jax import lax
from jax.experimental import pallas as pl
from jax.experimental.pallas import tpu as pltpu
from jax.experimental.pallas import tpu_sc as plsc

_H_STACK = 128


def _tc_body(sp_smem, stack_ref, hid_ref, out_ref):
    b = pl.program_id(0)
    tgt = sp_smem[b] + 1
    out_ref[...] = stack_ref[...]
    out_ref[0, pl.ds(tgt, 1)] = hid_ref[...]


def _make_ptr_kernel(B):
    mesh = plsc.VectorSubcoreMesh(core_axis_name="c", subcore_axis_name="s")

    @functools.partial(
        pl.kernel,
        mesh=mesh,
        compiler_params=pltpu.CompilerParams(needs_layout_passes=False),
        out_type=jax.ShapeDtypeStruct((B,), jnp.int32),
        scratch_types=[
            pltpu.VMEM((B,), jnp.int32),
            pltpu.VMEM((B,), jnp.float32),
            pltpu.VMEM((B,), jnp.float32),
            pltpu.VMEM((B,), jnp.float32),
            pltpu.VMEM((B,), jnp.int32),
        ],
    )
    def _sc(sp_hbm, op0_hbm, op1_hbm, op2_hbm, ptr_hbm,
            sp_v, x0_v, x1_v, x2_v, ptr_v):
        c = lax.axis_index("c")
        s = lax.axis_index("s")

        @pl.when((c == 0) & (s == 0))
        def _pointers():
            pltpu.sync_copy(sp_hbm, sp_v)
            pltpu.sync_copy(op0_hbm, x0_v)
            pltpu.sync_copy(op1_hbm, x1_v)
            pltpu.sync_copy(op2_hbm, x2_v)
            x0, x1, x2 = x0_v[...], x1_v[...], x2_v[...]
            ops = jnp.where((x0 >= x1) & (x0 >= x2), 0,
                            jnp.where(x1 >= x2, 1, 2)).astype(jnp.int32)
            ptr_v[...] = jnp.maximum(sp_v[...] + ops - 1, 0)
            pltpu.sync_copy(ptr_v, ptr_hbm)

    return _sc


def kernel(stack, stack_pointers, stack_op, hiddens):
    B, T1, N, H = stack.shape
    sp_i32 = stack_pointers.astype(jnp.int32)

    grid_spec = pltpu.PrefetchScalarGridSpec(
        num_scalar_prefetch=1,
        grid=(B,),
        in_specs=[
            pl.BlockSpec((1, T1, N, H), lambda b, sp: (b, 0, 0, 0)),
            pl.BlockSpec((1, N, _H_STACK), lambda b, sp: (b, 0, 0)),
        ],
        out_specs=pl.BlockSpec((1, T1, N, H), lambda b, sp: (b, 0, 0, 0)),
    )

    new_stack = pl.pallas_call(
        _tc_body,
        grid_spec=grid_spec,
        out_shape=jax.ShapeDtypeStruct((B, T1, N, H), stack.dtype),
    )(sp_i32, stack, hiddens)

    ptr_kernel = _make_ptr_kernel(B)
    new_ptr = ptr_kernel(sp_i32, stack_op[:, 0], stack_op[:, 1],
                         stack_op[:, 2])

    return new_stack, new_ptr.astype(stack_pointers.dtype)


# speedup vs baseline: 1.1235x; 1.1235x over previous
import jax
import jax.numpy as jnp
from jax.experimental import pallas as pl
from jax.experimental.pallas import tpu as pltpu

_H_STACK = 128


def _body(sp_smem, stack_ref, hid_ref, sp_vec_ref, op_ref, out_ref, ptr_ref,
          sem):
    b = pl.program_id(0)
    tgt = sp_smem[b] + 1

    cp = pltpu.make_async_copy(stack_ref, out_ref, sem)
    cp.start()
    cp.wait()
    out_ref[0, pl.ds(tgt, 1)] = hid_ref[...]

    @pl.when(b == 0)
    def _pointers():
        x0 = op_ref[:, 0:1]
        x1 = op_ref[:, 1:2]
        x2 = op_ref[:, 2:3]
        ops = jnp.where((x0 >= x1) & (x0 >= x2), 0,
                        jnp.where(x1 >= x2, 1, 2)).astype(jnp.int32)
        ptr_ref[...] = jnp.maximum(sp_vec_ref[...] + ops - 1, 0)


def kernel(stack, stack_pointers, stack_op, hiddens):
    B, T1, N, H = stack.shape
    sp_i32 = stack_pointers.astype(jnp.int32)

    grid_spec = pltpu.PrefetchScalarGridSpec(
        num_scalar_prefetch=1,
        grid=(B,),
        in_specs=[
            pl.BlockSpec((1, T1, N, H), lambda b, sp: (b, 0, 0, 0)),
            pl.BlockSpec((1, N, _H_STACK), lambda b, sp: (b, 0, 0)),
            pl.BlockSpec((B, 1), lambda b, sp: (0, 0)),
            pl.BlockSpec((B, 3), lambda b, sp: (0, 0)),
        ],
        out_specs=[
            pl.BlockSpec((1, T1, N, H), lambda b, sp: (b, 0, 0, 0)),
            pl.BlockSpec((B, 1), lambda b, sp: (0, 0)),
        ],
        scratch_shapes=[pltpu.SemaphoreType.DMA],
    )

    new_stack, new_ptr = pl.pallas_call(
        _body,
        grid_spec=grid_spec,
        out_shape=[
            jax.ShapeDtypeStruct((B, T1, N, H), stack.dtype),
            jax.ShapeDtypeStruct((B, 1), jnp.int32),
        ],
    )(sp_i32, stack, hiddens, sp_i32.reshape(B, 1), stack_op)

    return new_stack, new_ptr.reshape(B).astype(stack_pointers.dtype)
